# baseline (speedup 1.0000x reference)
import functools
import math

import jax
import jax.numpy as jnp
from jax import lax
from jax.experimental import pallas as pl
from jax.experimental.pallas import tpu as pltpu
from jax.experimental.pallas import tpu_sc as plsc

D_MODEL = 64
SCALE = math.sqrt(D_MODEL)

_info = plsc.get_sparse_core_info()
_NC, _NS, _L = _info.num_cores, _info.num_subcores, _info.num_lanes
_NW = _NC * _NS

_BB = 128


def _make_kernel(BATCH: int, T: int):
  assert BATCH == _NW * _BB
  n_jh = D_MODEL // 8
  mesh = plsc.VectorSubcoreMesh(core_axis_name="c", subcore_axis_name="s")

  @functools.partial(
      pl.kernel,
      mesh=mesh,
      compiler_params=pltpu.CompilerParams(use_tc_tiling_on_sc=False,
                                           needs_layout_passes=False),
      out_type=jax.ShapeDtypeStruct((T, n_jh, _NW, 8, _L * 8), jnp.float32),
      scratch_types=[
          pltpu.VMEM((_BB * T,), jnp.int32),
          pltpu.VMEM((2, _BB), jnp.int32),
          pltpu.VMEM((2, _BB, D_MODEL), jnp.float32),
          pltpu.VMEM((2, n_jh, 8, _BB), jnp.float32),
          pltpu.SemaphoreType.DMA,
          pltpu.SemaphoreType.DMA,
          pltpu.SemaphoreType.DMA,
          pltpu.SemaphoreType.DMA,
      ],
  )
  def gather_kernel(table_hbm, idx_hbm, out_hbm, idx_all, idx_t, gbuf, sbuf,
                    sem_g0, sem_g1, sem_s0, sem_s1):
    wid = lax.axis_index("s") * _NC + lax.axis_index("c")
    sem_g = (sem_g0, sem_g1)
    sem_s = (sem_s0, sem_s1)

    lane = lax.iota(jnp.int32, _L)
    rowsel = [lane + ilb * _L for ilb in range(_BB // _L)]
    colsel = [lane * T + ilb * _L * T for ilb in range(_BB // _L)]

    def build_idx(t, b):
      vals = [plsc.load_gather(idx_all, [colsel[ilb] + t])
              for ilb in range(_BB // _L)]
      for ilb in range(_BB // _L):
        idx_t[b, pl.ds(ilb * _L, _L)] = vals[ilb]

    def fire_gather(b):
      pltpu.async_copy(table_hbm.at[idx_t.at[b]], gbuf.at[b], sem_g[b])

    def wait_gather(b):
      pltpu.make_async_copy(table_hbm.at[idx_t.at[b]], gbuf.at[b],
                            sem_g[b]).wait()

    def transpose_scale(b):
      g_ref = gbuf.at[b]
      s_ref = sbuf.at[b]

      @plsc.parallel_loop(0, D_MODEL, unroll=2)
      def _(f):
        fvec = jnp.full((_L,), f, jnp.int32)
        vals = [plsc.load_gather(g_ref, [rowsel[ilb], fvec]) * SCALE
                for ilb in range(_BB // _L)]
        jh = lax.shift_right_logical(f, 3)
        jl = lax.bitwise_and(f, 7)
        for ilb in range(_BB // _L):
          s_ref[jh, jl, pl.ds(ilb * _L, _L)] = vals[ilb]

    def fire_store(t, b):
      for jh in range(n_jh):
        pltpu.async_copy(sbuf.at[b, jh], out_hbm.at[t, jh, wid], sem_s[b])

    def wait_store(t, b):
      for jh in range(n_jh):
        pltpu.make_async_copy(sbuf.at[b, jh], out_hbm.at[t, jh, wid],
                              sem_s[b]).wait()

    pltpu.sync_copy(idx_hbm.at[pl.ds(wid * _BB * T, _BB * T)], idx_all)
    build_idx(0, 0)
    fire_gather(0)

    def pair_body(p, carry):
      for b in range(2):
        t = 2 * p + b
        nb = 1 - b

        @pl.when(t + 1 < T)
        def _():
          build_idx(t + 1, nb)
          fire_gather(nb)

        wait_gather(b)

        @pl.when(t >= 2)
        def _():
          wait_store(t - 2, b)

        transpose_scale(b)
        fire_store(t, b)

      return carry

    lax.fori_loop(0, T // 2, pair_body, 0)
    wait_store(T - 2, 0)
    wait_store(T - 1, 1)

  return gather_kernel


def kernel(x, table):
  BATCH, T = x.shape
  idx = x.reshape(-1).astype(jnp.int32)
  out5 = _make_kernel(BATCH, T)(table, idx)
  out = out5.transpose(2, 4, 0, 1, 3).reshape(BATCH, T, D_MODEL)
  return out


# speedup vs baseline: 1.4234x; 1.4234x over previous
import functools
import math

import jax
import jax.numpy as jnp
from jax import lax
from jax.experimental import pallas as pl
from jax.experimental.pallas import tpu as pltpu
from jax.experimental.pallas import tpu_sc as plsc

D_MODEL = 64
SCALE = math.sqrt(D_MODEL)

_info = plsc.get_sparse_core_info()
_NC, _NS, _L = _info.num_cores, _info.num_subcores, _info.num_lanes
_NW = _NC * _NS

_BB = 128


def _make_kernel(BATCH: int, T: int):
  assert BATCH == _NW * _BB
  n_jh = D_MODEL // 8
  mesh = plsc.VectorSubcoreMesh(core_axis_name="c", subcore_axis_name="s")

  @functools.partial(
      pl.kernel,
      mesh=mesh,
      compiler_params=pltpu.CompilerParams(use_tc_tiling_on_sc=False,
                                           needs_layout_passes=False),
      out_type=jax.ShapeDtypeStruct((T, n_jh, _NW, 8, _L * 8), jnp.float32),
      scratch_types=[
          pltpu.VMEM((T, _BB), jnp.int32),
          pltpu.VMEM((2, _BB, D_MODEL), jnp.float32),
          pltpu.VMEM((2, D_MODEL, _BB), jnp.float32),
          pltpu.SemaphoreType.DMA,
          pltpu.SemaphoreType.DMA,
          pltpu.SemaphoreType.DMA,
          pltpu.SemaphoreType.DMA,
      ],
  )
  def gather_kernel(table_hbm, idx_hbm, out_hbm, idx_all, gbuf, sbuf,
                    sem_g0, sem_g1, sem_s0, sem_s1):
    wid = lax.axis_index("s") * _NC + lax.axis_index("c")
    sem_g = (sem_g0, sem_g1)
    sem_s = (sem_s0, sem_s1)

    lane = lax.iota(jnp.int32, _L)
    rot = [lax.rem(lane + k, _L) for k in range(_L)]

    def fire_gather(t, b):
      pltpu.async_copy(table_hbm.at[idx_all.at[t]], gbuf.at[b], sem_g[b])

    def wait_gather(t, b):
      pltpu.make_async_copy(table_hbm.at[idx_all.at[t]], gbuf.at[b],
                            sem_g[b]).wait()

    def transpose_scale(b):
      g_ref = gbuf.at[b]
      s_ref = sbuf.at[b]

      @plsc.parallel_loop(0, D_MODEL // _L * (_BB // _L))
      def _(i):
        fb = lax.shift_right_logical(i, 3) * _L
        ilb = lax.bitwise_and(i, 7)
        row_vec = lane + ilb * _L
        for k in range(_L):
          feat_vec = rot[k] + fb
          v = plsc.load_gather(g_ref, [row_vec, feat_vec])
          plsc.store_scatter(s_ref, [feat_vec, row_vec], v * SCALE)

    def fire_store(t, b):
      for jh in range(n_jh):
        pltpu.async_copy(sbuf.at[b, pl.ds(jh * 8, 8)],
                         out_hbm.at[t, jh, wid], sem_s[b])

    def wait_store(t, b):
      for jh in range(n_jh):
        pltpu.make_async_copy(sbuf.at[b, pl.ds(jh * 8, 8)],
                              out_hbm.at[t, jh, wid], sem_s[b]).wait()

    pltpu.sync_copy(idx_hbm.at[:, pl.ds(wid * _BB, _BB)], idx_all)
    fire_gather(0, 0)

    def pair_body(p, carry):
      for b in range(2):
        t = 2 * p + b
        nb = 1 - b

        @pl.when(t + 1 < T)
        def _():
          fire_gather(t + 1, nb)

        wait_gather(t, b)

        @pl.when(t >= 2)
        def _():
          wait_store(t - 2, b)

        transpose_scale(b)
        fire_store(t, b)

      return carry

    lax.fori_loop(0, T // 2, pair_body, 0)
    wait_store(T - 2, 0)
    wait_store(T - 1, 1)

  return gather_kernel


def kernel(x, table):
  BATCH, T = x.shape
  idx_tmajor = x.T.astype(jnp.int32)
  out5 = _make_kernel(BATCH, T)(table, idx_tmajor)
  out = out5.transpose(2, 4, 0, 1, 3).reshape(BATCH, T, D_MODEL)
  return out


# speedup vs baseline: 1.4279x; 1.0032x over previous
import functools
import math

import jax
import jax.numpy as jnp
from jax import lax
from jax.experimental import pallas as pl
from jax.experimental.pallas import tpu as pltpu
from jax.experimental.pallas import tpu_sc as plsc

D_MODEL = 64
SCALE = math.sqrt(D_MODEL)

_info = plsc.get_sparse_core_info()
_NC, _NS, _L = _info.num_cores, _info.num_subcores, _info.num_lanes
_NW = _NC * _NS

_BB = 128


def _make_kernel(BATCH: int, T: int):
  assert BATCH == _NW * _BB
  n_jh = D_MODEL // 8
  mesh = plsc.VectorSubcoreMesh(core_axis_name="c", subcore_axis_name="s")

  @functools.partial(
      pl.kernel,
      mesh=mesh,
      compiler_params=pltpu.CompilerParams(use_tc_tiling_on_sc=False,
                                           needs_layout_passes=False),
      out_type=jax.ShapeDtypeStruct((T, n_jh, _NW, 8, _L * 8), jnp.float32),
      scratch_types=[
          pltpu.VMEM((T // 8, 8, _BB), jnp.int32),
          pltpu.VMEM((2, _BB, D_MODEL), jnp.float32),
          pltpu.VMEM((2, D_MODEL, _BB), jnp.float32),
          pltpu.SemaphoreType.DMA,
          pltpu.SemaphoreType.DMA,
          pltpu.SemaphoreType.DMA,
          pltpu.SemaphoreType.DMA,
      ],
  )
  def gather_kernel(table_hbm, idx_hbm, out_hbm, idx_all, gbuf, sbuf,
                    sem_g0, sem_g1, sem_s0, sem_s1):
    wid = lax.axis_index("s") * _NC + lax.axis_index("c")
    sem_g = (sem_g0, sem_g1)
    sem_s = (sem_s0, sem_s1)

    lane = lax.iota(jnp.int32, _L)
    rot = [lax.rem(lane + k, _L) for k in range(_L)]

    def idx_row(t):
      return idx_all.at[lax.shift_right_logical(t, 3), lax.bitwise_and(t, 7)]

    def fire_gather(t, b):
      pltpu.async_copy(table_hbm.at[idx_row(t)], gbuf.at[b], sem_g[b])

    def wait_gather(t, b):
      pltpu.make_async_copy(table_hbm.at[idx_row(t)], gbuf.at[b],
                            sem_g[b]).wait()

    def transpose_scale(b):
      g_ref = gbuf.at[b]
      s_ref = sbuf.at[b]

      @plsc.parallel_loop(0, D_MODEL // _L * (_BB // _L))
      def _(i):
        fb = lax.shift_right_logical(i, 3) * _L
        ilb = lax.bitwise_and(i, 7)
        row_vec = lane + ilb * _L
        for k in range(_L):
          feat_vec = rot[k] + fb
          v = plsc.load_gather(g_ref, [row_vec, feat_vec])
          plsc.store_scatter(s_ref, [feat_vec, row_vec], v * SCALE)

    def fire_store(t, b):
      for jh in range(n_jh):
        pltpu.async_copy(sbuf.at[b, pl.ds(jh * 8, 8)],
                         out_hbm.at[t, jh, wid], sem_s[b])

    def wait_store(t, b):
      for jh in range(n_jh):
        pltpu.make_async_copy(sbuf.at[b, pl.ds(jh * 8, 8)],
                              out_hbm.at[t, jh, wid], sem_s[b]).wait()

    pltpu.sync_copy(idx_hbm.at[:, wid], idx_all)
    fire_gather(0, 0)

    def pair_body(p, carry):
      for b in range(2):
        t = 2 * p + b
        nb = 1 - b

        @pl.when(t + 1 < T)
        def _():
          fire_gather(t + 1, nb)

        wait_gather(t, b)

        @pl.when(t >= 2)
        def _():
          wait_store(t - 2, b)

        transpose_scale(b)
        fire_store(t, b)

      return carry

    lax.fori_loop(0, T // 2, pair_body, 0)
    wait_store(T - 2, 0)
    wait_store(T - 1, 1)

  return gather_kernel


def kernel(x, table):
  BATCH, T = x.shape
  idx4 = (x.astype(jnp.int32)
          .reshape(BATCH // _BB, _BB, T // 8, 8)
          .transpose(2, 0, 3, 1))
  out5 = _make_kernel(BATCH, T)(table, idx4)
  out = out5.transpose(2, 4, 0, 1, 3).reshape(BATCH, T, D_MODEL)
  return out


# speedup vs baseline: 1.7979x; 1.2591x over previous
import functools
import math

import jax
import jax.numpy as jnp
from jax import lax
from jax.experimental import pallas as pl
from jax.experimental.pallas import tpu as pltpu
from jax.experimental.pallas import tpu_sc as plsc

D_MODEL = 64
SCALE = math.sqrt(D_MODEL)

_info = plsc.get_sparse_core_info()
_NC, _NS, _L = _info.num_cores, _info.num_subcores, _info.num_lanes
_NW = _NC * _NS

_BB = 128


def _make_kernel(BATCH: int, T: int):
  assert BATCH == _NW * _BB
  n_jh = D_MODEL // 8
  mesh = plsc.VectorSubcoreMesh(core_axis_name="c", subcore_axis_name="s")

  @functools.partial(
      pl.kernel,
      mesh=mesh,
      compiler_params=pltpu.CompilerParams(use_tc_tiling_on_sc=False,
                                           needs_layout_passes=False),
      out_type=jax.ShapeDtypeStruct((T, n_jh, _NW, 8, _L * 8), jnp.float32),
      scratch_types=[
          pltpu.VMEM((T // 8, 8, _BB), jnp.int32),
          pltpu.VMEM((4, _BB, D_MODEL), jnp.float32),
          pltpu.VMEM((4, D_MODEL, _BB), jnp.float32),
          pltpu.SemaphoreType.DMA,
          pltpu.SemaphoreType.DMA,
          pltpu.SemaphoreType.DMA,
          pltpu.SemaphoreType.DMA,
          pltpu.SemaphoreType.DMA,
          pltpu.SemaphoreType.DMA,
          pltpu.SemaphoreType.DMA,
          pltpu.SemaphoreType.DMA,
      ],
  )
  def gather_kernel(table_hbm, idx_hbm, out_hbm, idx_all, gbuf, sbuf,
                    sem_g0, sem_g1, sem_g2, sem_g3,
                    sem_s0, sem_s1, sem_s2, sem_s3):
    wid = lax.axis_index("s") * _NC + lax.axis_index("c")
    sem_g = (sem_g0, sem_g1, sem_g2, sem_g3)
    sem_s = (sem_s0, sem_s1, sem_s2, sem_s3)

    lane = lax.iota(jnp.int32, _L)
    rot = [lax.rem(lane + k, _L) for k in range(_L)]

    def idx_row(t):
      return idx_all.at[lax.shift_right_logical(t, 3), lax.bitwise_and(t, 7)]

    def fire_gather(t, b):
      pltpu.async_copy(table_hbm.at[idx_row(t)], gbuf.at[b], sem_g[b])

    def wait_gather(t, b):
      pltpu.make_async_copy(table_hbm.at[idx_row(t)], gbuf.at[b],
                            sem_g[b]).wait()

    def transpose_scale(b):
      g_ref = gbuf.at[b]
      s_ref = sbuf.at[b]

      @plsc.parallel_loop(0, D_MODEL // _L * (_BB // _L), unroll=2)
      def _(i):
        fb = lax.shift_right_logical(i, 3) * _L
        ilb = lax.bitwise_and(i, 7)
        row_vec = lane + ilb * _L
        for k in range(_L):
          feat_vec = rot[k] + fb
          v = plsc.load_gather(g_ref, [row_vec, feat_vec])
          plsc.store_scatter(s_ref, [feat_vec, row_vec], v * SCALE)

    def fire_store(t, b):
      for jh in range(n_jh):
        pltpu.async_copy(sbuf.at[b, pl.ds(jh * 8, 8)],
                         out_hbm.at[t, jh, wid], sem_s[b])

    def wait_store(t, b):
      for jh in range(n_jh):
        pltpu.make_async_copy(sbuf.at[b, pl.ds(jh * 8, 8)],
                              out_hbm.at[t, jh, wid], sem_s[b]).wait()

    pltpu.sync_copy(idx_hbm.at[:, wid], idx_all)
    fire_gather(0, 0)
    fire_gather(1, 1)
    fire_gather(2, 2)

    def quad_body(p, carry):
      for b in range(4):
        t = 4 * p + b
        nb = (b + 3) % 4

        @pl.when(t + 3 < T)
        def _():
          fire_gather(t + 3, nb)

        wait_gather(t, b)

        @pl.when(t >= 4)
        def _():
          wait_store(t - 4, b)

        transpose_scale(b)
        fire_store(t, b)

      return carry

    lax.fori_loop(0, T // 4, quad_body, 0)
    for b in range(4):
      wait_store(T - 4 + b, b)

  return gather_kernel


def kernel(x, table):
  BATCH, T = x.shape
  idx4 = (x.astype(jnp.int32)
          .reshape(BATCH // _BB, _BB, T // 8, 8)
          .transpose(2, 0, 3, 1))
  out5 = _make_kernel(BATCH, T)(table, idx4)
  out = out5.transpose(2, 4, 0, 1, 3).reshape(BATCH, T, D_MODEL)
  return out


# speedup vs baseline: 1.8425x; 1.0248x over previous
import functools
import math

import jax
import jax.numpy as jnp
from jax import lax
from jax.experimental import pallas as pl
from jax.experimental.pallas import tpu as pltpu
from jax.experimental.pallas import tpu_sc as plsc

D_MODEL = 64
SCALE = math.sqrt(D_MODEL)

_info = plsc.get_sparse_core_info()
_NC, _NS, _L = _info.num_cores, _info.num_subcores, _info.num_lanes
_NW = _NC * _NS

_BB = 128


def _make_kernel(BATCH: int, T: int):
  assert BATCH == _NW * _BB
  n_jh = D_MODEL // 8
  mesh = plsc.VectorSubcoreMesh(core_axis_name="c", subcore_axis_name="s")

  @functools.partial(
      pl.kernel,
      mesh=mesh,
      compiler_params=pltpu.CompilerParams(use_tc_tiling_on_sc=False,
                                           needs_layout_passes=False),
      out_type=jax.ShapeDtypeStruct((T, n_jh, _NW, 8, _L * 8), jnp.float32),
      scratch_types=[
          pltpu.VMEM((T // 8, 8, _BB), jnp.int32),
          pltpu.VMEM((4, _BB, D_MODEL), jnp.float32),
          pltpu.VMEM((4, D_MODEL, _BB), jnp.float32),
          pltpu.SemaphoreType.DMA,
          pltpu.SemaphoreType.DMA,
          pltpu.SemaphoreType.DMA,
          pltpu.SemaphoreType.DMA,
          pltpu.SemaphoreType.DMA,
          pltpu.SemaphoreType.DMA,
          pltpu.SemaphoreType.DMA,
          pltpu.SemaphoreType.DMA,
      ],
  )
  def gather_kernel(table_hbm, idx_hbm, out_hbm, idx_all, gbuf, sbuf,
                    sem_g0, sem_g1, sem_g2, sem_g3,
                    sem_s0, sem_s1, sem_s2, sem_s3):
    wid = lax.axis_index("s") * _NC + lax.axis_index("c")
    sem_g = (sem_g0, sem_g1, sem_g2, sem_g3)
    sem_s = (sem_s0, sem_s1, sem_s2, sem_s3)

    lane = lax.iota(jnp.int32, _L)
    rot = [lax.rem(lane + k, _L) for k in range(_L)]

    def idx_row(t):
      return idx_all.at[lax.shift_right_logical(t, 3), lax.bitwise_and(t, 7)]

    def fire_gather(t, b):
      pltpu.async_copy(table_hbm.at[idx_row(t)], gbuf.at[b], sem_g[b])

    def wait_gather(t, b):
      pltpu.make_async_copy(table_hbm.at[idx_row(t)], gbuf.at[b],
                            sem_g[b]).wait()

    def transpose_scale(b):
      g_ref = gbuf.at[b]
      s_ref = sbuf.at[b]

      @plsc.parallel_loop(0, D_MODEL // _L * (_BB // _L), unroll=4)
      def _(i):
        fb = lax.shift_right_logical(i, 3) * _L
        ilb = lax.bitwise_and(i, 7)
        row_vec = lane + ilb * _L
        for k in range(_L):
          feat_vec = rot[k] + fb
          v = plsc.load_gather(g_ref, [row_vec, feat_vec])
          plsc.store_scatter(s_ref, [feat_vec, row_vec], v * SCALE)

    def fire_store(t, b):
      for jh in range(n_jh):
        pltpu.async_copy(sbuf.at[b, pl.ds(jh * 8, 8)],
                         out_hbm.at[t, jh, wid], sem_s[b])

    def wait_store(t, b):
      for jh in range(n_jh):
        pltpu.make_async_copy(sbuf.at[b, pl.ds(jh * 8, 8)],
                              out_hbm.at[t, jh, wid], sem_s[b]).wait()

    pltpu.sync_copy(idx_hbm.at[:, wid], idx_all)
    fire_gather(0, 0)
    fire_gather(1, 1)
    fire_gather(2, 2)

    def quad_body(p, carry):
      for b in range(4):
        t = 4 * p + b
        nb = (b + 3) % 4

        @pl.when(t + 3 < T)
        def _():
          fire_gather(t + 3, nb)

        wait_gather(t, b)

        @pl.when(t >= 4)
        def _():
          wait_store(t - 4, b)

        transpose_scale(b)
        fire_store(t, b)

      return carry

    lax.fori_loop(0, T // 4, quad_body, 0)
    for b in range(4):
      wait_store(T - 4 + b, b)

  return gather_kernel


def kernel(x, table):
  BATCH, T = x.shape
  idx4 = (x.astype(jnp.int32)
          .reshape(BATCH // _BB, _BB, T // 8, 8)
          .transpose(2, 0, 3, 1))
  out5 = _make_kernel(BATCH, T)(table, idx4)
  out = out5.transpose(2, 4, 0, 1, 3).reshape(BATCH, T, D_MODEL)
  return out
